# baseline (speedup 1.0000x reference)
import functools

import jax
import jax.numpy as jnp
from jax import lax
from jax.experimental import pallas as pl
from jax.experimental.pallas import tpu as pltpu


def _lightgcn_kernel(nu, n, np_, br, f_ref, w_ref, b2_ref, emb_ref, a_ref,
                     out_ref, e_a, e_b, acc):
    l = pl.program_id(0)
    r = pl.program_id(1)
    rows = pl.ds(r * br, br)

    @pl.when(l == 0)
    def _prologue():
        feat = jnp.dot(f_ref[...], w_ref[...],
                       preferred_element_type=jnp.float32)
        gr = r * br + lax.broadcasted_iota(jnp.int32, (br, 1), 0)
        bias = jnp.where(gr < nu, b2_ref[0:1, :], b2_ref[1:2, :])
        res = emb_ref[...] + feat + bias
        e_a[rows, :] = res
        acc[rows, :] = res

    @pl.when(l == 1)
    def _layer1():
        res = jnp.dot(a_ref[...], e_a[pl.ds(0, n), :],
                      preferred_element_type=jnp.float32)
        e_b[rows, :] = res
        acc[rows, :] = acc[rows, :] + res

    @pl.when(l == 2)
    def _layer2():
        res = jnp.dot(a_ref[...], e_b[pl.ds(0, n), :],
                      preferred_element_type=jnp.float32)
        e_a[rows, :] = res
        acc[rows, :] = acc[rows, :] + res

    @pl.when(l == 3)
    def _layer3():
        res = jnp.dot(a_ref[...], e_a[pl.ds(0, n), :],
                      preferred_element_type=jnp.float32)
        out_ref[...] = (acc[rows, :] + res) * 0.25


@jax.jit
def kernel(adj, user_features, item_features, user_emb, item_emb, Wu, bu, Wi,
           bi):
    n = adj.shape[0]
    nu, fu = user_features.shape
    ni, fi = item_features.shape
    emb = user_emb.shape[1]
    br = 128
    nb = -(-n // br)
    np_ = nb * br
    fk = fu + fi

    f = jnp.zeros((n, fk), jnp.float32)
    f = f.at[:nu, :fu].set(user_features)
    f = f.at[nu:, fu:].set(item_features)
    w = jnp.concatenate([Wu, Wi], axis=0)
    emb0 = jnp.concatenate([user_emb, item_emb], axis=0)
    b2 = jnp.zeros((8, emb), jnp.float32).at[0].set(bu).at[1].set(bi)

    grid = (4, nb)
    body = functools.partial(_lightgcn_kernel, nu, n, np_, br)
    out = pl.pallas_call(
        body,
        grid=grid,
        in_specs=[
            pl.BlockSpec((br, fk), lambda l, r: (jnp.where(l == 0, r, 0), 0)),
            pl.BlockSpec((fk, emb), lambda l, r: (0, 0)),
            pl.BlockSpec((8, emb), lambda l, r: (0, 0)),
            pl.BlockSpec((br, emb), lambda l, r: (jnp.where(l == 0, r, 0), 0)),
            pl.BlockSpec((br, n), lambda l, r: (jnp.where(l == 0, 0, r), 0)),
        ],
        out_specs=pl.BlockSpec((br, emb), lambda l, r: (r, 0)),
        out_shape=jax.ShapeDtypeStruct((n, emb), jnp.float32),
        scratch_shapes=[
            pltpu.VMEM((np_, emb), jnp.float32),
            pltpu.VMEM((np_, emb), jnp.float32),
            pltpu.VMEM((np_, emb), jnp.float32),
        ],
        compiler_params=pltpu.CompilerParams(
            dimension_semantics=("arbitrary", "arbitrary")),
    )(f, w, b2, emb0, adj)

    return out[:nu], out[nu:]


# speedup vs baseline: 1.1951x; 1.1951x over previous
import functools

import jax
import jax.numpy as jnp
from jax import lax
from jax.experimental import pallas as pl
from jax.experimental.pallas import tpu as pltpu


def _lightgcn_kernel(nu, n, np_, br, f_ref, w_ref, b2_ref, emb_ref, a_ref,
                     out_ref, e_a, e_b, acc):
    l = pl.program_id(0)
    r = pl.program_id(1)
    rows = pl.ds(r * br, br)

    @pl.when(l == 0)
    def _prologue():
        feat = jnp.dot(f_ref[...], w_ref[...],
                       preferred_element_type=jnp.float32)
        gr = r * br + lax.broadcasted_iota(jnp.int32, (br, 1), 0)
        bias = jnp.where(gr < nu, b2_ref[0:1, :], b2_ref[1:2, :])
        res = emb_ref[...] + feat + bias
        e_a[rows, :] = res
        acc[rows, :] = res

    @pl.when(l == 1)
    def _layer1():
        res = jnp.dot(a_ref[...], e_a[pl.ds(0, n), :],
                      preferred_element_type=jnp.float32)
        e_b[rows, :] = res
        acc[rows, :] = acc[rows, :] + res

    @pl.when(l == 2)
    def _layer2():
        res = jnp.dot(a_ref[...], e_b[pl.ds(0, n), :],
                      preferred_element_type=jnp.float32)
        e_a[rows, :] = res
        acc[rows, :] = acc[rows, :] + res

    @pl.when(l == 3)
    def _layer3():
        res = jnp.dot(a_ref[...], e_a[pl.ds(0, n), :],
                      preferred_element_type=jnp.float32)
        out_ref[...] = (acc[rows, :] + res) * 0.25


@jax.jit
def kernel(adj, user_features, item_features, user_emb, item_emb, Wu, bu, Wi,
           bi):
    n = adj.shape[0]
    nu, fu = user_features.shape
    ni, fi = item_features.shape
    emb = user_emb.shape[1]
    br = 256
    nb = -(-n // br)
    np_ = nb * br
    fk = fu + fi

    f = jnp.zeros((n, fk), jnp.float32)
    f = f.at[:nu, :fu].set(user_features)
    f = f.at[nu:, fu:].set(item_features)
    w = jnp.concatenate([Wu, Wi], axis=0)
    emb0 = jnp.concatenate([user_emb, item_emb], axis=0)
    b2 = jnp.zeros((8, emb), jnp.float32).at[0].set(bu).at[1].set(bi)

    grid = (4, nb)
    body = functools.partial(_lightgcn_kernel, nu, n, np_, br)
    out = pl.pallas_call(
        body,
        grid=grid,
        in_specs=[
            pl.BlockSpec((br, fk), lambda l, r: (jnp.where(l == 0, r, 0), 0)),
            pl.BlockSpec((fk, emb), lambda l, r: (0, 0)),
            pl.BlockSpec((8, emb), lambda l, r: (0, 0)),
            pl.BlockSpec((br, emb), lambda l, r: (jnp.where(l == 0, r, 0), 0)),
            pl.BlockSpec((br, n), lambda l, r: (jnp.where(l == 0, 0, r), 0)),
        ],
        out_specs=pl.BlockSpec((br, emb), lambda l, r: (r, 0)),
        out_shape=jax.ShapeDtypeStruct((n, emb), jnp.float32),
        scratch_shapes=[
            pltpu.VMEM((np_, emb), jnp.float32),
            pltpu.VMEM((np_, emb), jnp.float32),
            pltpu.VMEM((np_, emb), jnp.float32),
        ],
        compiler_params=pltpu.CompilerParams(
            dimension_semantics=("arbitrary", "arbitrary")),
    )(f, w, b2, emb0, adj)

    return out[:nu], out[nu:]
